# Initial kernel scaffold; baseline (speedup 1.0000x reference)
import jax
import jax.numpy as jnp
from jax.experimental import pallas as pl


def kernel(x, edge_index, Wd0, bd0, Wd1, bd1, Wd2, bd2, Wd3, bd3, p1, p2, p3, Wu0, bu0, Wu1, bu1, Wu2, bu2, Wo, bo):
    raise NotImplementedError("write your pallas kernel here")


# speedup vs baseline: 1.9921x; 1.9921x over previous
import functools
from math import ceil

import jax
import jax.numpy as jnp
from jax import lax
from jax.experimental import pallas as pl
from jax.experimental.pallas import tpu as pltpu
from jax.experimental.pallas import tpu_sc as plsc

RATIOS = (0.8, 0.7, 0.6)
SPREAD = 128
BN = 512
BC = 512
E_PAD = 163840
WIN = 64


def _pad_rows(n):
    return ((n + SPREAD + 511) // 512) * 512


def _mm_split(x, W, dinv, gate, n_valid):
    Np, K = x.shape
    nb = Np // BN

    def body(x_ref, w_ref, d_ref, g_ref, o_ref):
        i = pl.program_id(1)
        z = jnp.dot(x_ref[...], w_ref[...], preferred_element_type=jnp.float32)
        rows = i * BN + lax.broadcasted_iota(jnp.int32, (BN, 1), 0)
        m = (rows < n_valid).astype(jnp.float32)
        o_ref[...] = z * jax.nn.sigmoid(g_ref[...]) * d_ref[...] * m

    return pl.pallas_call(
        body,
        grid=(2, nb),
        in_specs=[
            pl.BlockSpec((BN, K), lambda h, i: (i, 0)),
            pl.BlockSpec((K, 128), lambda h, i: (0, h)),
            pl.BlockSpec((BN, 1), lambda h, i: (i, 0)),
            pl.BlockSpec((BN, 1), lambda h, i: (i, 0)),
        ],
        out_specs=pl.BlockSpec((BN, 128), lambda h, i: (h * nb + i, 0)),
        out_shape=jax.ShapeDtypeStruct((2 * Np, 128), jnp.float32),
    )(x, W, dinv, gate)


def _mm_split_acc(x, W, acc, dinv, n_valid):
    Np, K = x.shape
    nb = Np // BN

    def body(x_ref, w_ref, a_ref, d_ref, o_ref):
        i = pl.program_id(1)
        z = jnp.dot(x_ref[...], w_ref[...], preferred_element_type=jnp.float32)
        rows = i * BN + lax.broadcasted_iota(jnp.int32, (BN, 1), 0)
        m = (rows < n_valid).astype(jnp.float32)
        o_ref[...] = (z + a_ref[...]) * d_ref[...] * m

    return pl.pallas_call(
        body,
        grid=(2, nb),
        in_specs=[
            pl.BlockSpec((BN, K), lambda h, i: (i, 0)),
            pl.BlockSpec((K, 128), lambda h, i: (0, h)),
            pl.BlockSpec((BN, 128), lambda h, i: (i, h)),
            pl.BlockSpec((BN, 1), lambda h, i: (i, 0)),
        ],
        out_specs=pl.BlockSpec((BN, 128), lambda h, i: (h * nb + i, 0)),
        out_shape=jax.ShapeDtypeStruct((2 * Np, 128), jnp.float32),
    )(x, W, acc, dinv)


def _mm_plain(x, W):
    Np, K = x.shape
    M = W.shape[1]
    nb = Np // BN

    def body(x_ref, w_ref, o_ref):
        o_ref[...] = jnp.dot(x_ref[...], w_ref[...],
                             preferred_element_type=jnp.float32)

    return pl.pallas_call(
        body,
        grid=(nb,),
        in_specs=[
            pl.BlockSpec((BN, K), lambda i: (i, 0)),
            pl.BlockSpec((K, M), lambda i: (0, 0)),
        ],
        out_specs=pl.BlockSpec((BN, M), lambda i: (i, 0)),
        out_shape=jax.ShapeDtypeStruct((Np, M), jnp.float32),
    )(x, W)


def _mm_scaled(x, W, dinv, n_valid):
    Np, K = x.shape
    M = W.shape[1]
    nb = Np // BN

    def body(x_ref, w_ref, d_ref, o_ref):
        i = pl.program_id(0)
        z = jnp.dot(x_ref[...], w_ref[...], preferred_element_type=jnp.float32)
        rows = i * BN + lax.broadcasted_iota(jnp.int32, (BN, 1), 0)
        m = (rows < n_valid).astype(jnp.float32)
        o_ref[...] = z * d_ref[...] * m

    return pl.pallas_call(
        body,
        grid=(nb,),
        in_specs=[
            pl.BlockSpec((BN, K), lambda i: (i, 0)),
            pl.BlockSpec((K, M), lambda i: (0, 0)),
            pl.BlockSpec((BN, 1), lambda i: (i, 0)),
        ],
        out_specs=pl.BlockSpec((BN, M), lambda i: (i, 0)),
        out_shape=jax.ShapeDtypeStruct((Np, M), jnp.float32),
    )(x, W, dinv)


def _post(y2, zs2, dinv, b, n_valid, relu=True):
    Np2 = y2.shape[0]
    Np = Np2 // 2
    nb = Np // BN
    b2 = b.reshape(1, 256)

    def body(yl, yr, zl, zr, d_ref, b_ref, o_ref):
        i = pl.program_id(0)
        rows = i * BN + lax.broadcasted_iota(jnp.int32, (BN, 1), 0)
        m = (rows < n_valid).astype(jnp.float32)
        d = d_ref[...]
        hl = d * (yl[...] + zl[...])
        hr = d * (yr[...] + zr[...])
        h = (jnp.concatenate([hl, hr], axis=1) + b_ref[...]) * m
        if relu:
            h = jnp.maximum(h, 0.0)
        o_ref[...] = h

    return pl.pallas_call(
        body,
        grid=(nb,),
        in_specs=[
            pl.BlockSpec((BN, 128), lambda i: (i, 0)),
            pl.BlockSpec((BN, 128), lambda i: (nb + i, 0)),
            pl.BlockSpec((BN, 128), lambda i: (i, 0)),
            pl.BlockSpec((BN, 128), lambda i: (nb + i, 0)),
            pl.BlockSpec((BN, 1), lambda i: (i, 0)),
            pl.BlockSpec((1, 256), lambda i: (0, 0)),
        ],
        out_specs=pl.BlockSpec((BN, 256), lambda i: (i, 0)),
        out_shape=jax.ShapeDtypeStruct((Np, 256), jnp.float32),
    )(y2, y2, zs2, zs2, dinv, b2)


def _post_final(y2, zs, dinv, b):
    Np2, M = y2.shape
    Np = Np2 // 2
    nb = Np // BN
    b2 = b.reshape(1, M)

    def body(y0, y1, z_ref, d_ref, b_ref, o_ref):
        o_ref[...] = d_ref[...] * (y0[...] + y1[...] + z_ref[...]) + b_ref[...]

    return pl.pallas_call(
        body,
        grid=(nb,),
        in_specs=[
            pl.BlockSpec((BN, M), lambda i: (i, 0)),
            pl.BlockSpec((BN, M), lambda i: (nb + i, 0)),
            pl.BlockSpec((BN, M), lambda i: (i, 0)),
            pl.BlockSpec((BN, 1), lambda i: (i, 0)),
            pl.BlockSpec((1, M), lambda i: (0, 0)),
        ],
        out_specs=pl.BlockSpec((BN, M), lambda i: (i, 0)),
        out_shape=jax.ShapeDtypeStruct((Np, M), jnp.float32),
    )(y2, y2, zs, dinv, b2)


def _dinv_from_deg(ydeg2):
    Np = ydeg2.shape[0] // 2
    nb = Np // BN

    def body(y0, y1, o_ref):
        c = y0[...][:, :1] + y1[...][:, :1]
        o_ref[...] = 1.0 / jnp.sqrt(1.0 + c)

    return pl.pallas_call(
        body,
        grid=(nb,),
        in_specs=[
            pl.BlockSpec((BN, 128), lambda i: (i, 0)),
            pl.BlockSpec((BN, 128), lambda i: (nb + i, 0)),
        ],
        out_specs=pl.BlockSpec((BN, 1), lambda i: (i, 0)),
        out_shape=jax.ShapeDtypeStruct((Np, 1), jnp.float32),
    )(ydeg2, ydeg2)


def _score(h, p, n_valid):
    Np = h.shape[0]
    nb = Np // BN
    p2 = p.reshape(256, 1)

    def body(h_ref, p_ref, o_ref):
        i = pl.program_id(0)
        pv = p_ref[...]
        nrm = jnp.sqrt(jnp.sum(pv * pv)) + 1e-12
        t = jnp.dot(h_ref[...], pv, preferred_element_type=jnp.float32) / nrm
        rows = i * BN + lax.broadcasted_iota(jnp.int32, (BN, 1), 0)
        o_ref[...] = jnp.where(rows < n_valid, t, -1e9)

    return pl.pallas_call(
        body,
        grid=(nb,),
        in_specs=[
            pl.BlockSpec((BN, 256), lambda i: (i, 0)),
            pl.BlockSpec((256, 1), lambda i: (0, 0)),
        ],
        out_specs=pl.BlockSpec((BN, 1), lambda i: (i, 0)),
        out_shape=jax.ShapeDtypeStruct((Np, 1), jnp.float32),
    )(h, p2)


def _rank_kept(t, k):
    Np = t.shape[0]
    nb = Np // BN
    nc = Np // BC
    tT = t.reshape(1, Np)

    def body(tc_ref, tr_ref, o_ref, acc_ref):
        i = pl.program_id(0)
        j = pl.program_id(1)

        @pl.when(j == 0)
        def _():
            acc_ref[...] = jnp.zeros_like(acc_ref)

        rows = i * BN + lax.broadcasted_iota(jnp.int32, (BN, 1), 0)
        cols = j * BC + lax.broadcasted_iota(jnp.int32, (1, BC), 1)
        tc = tc_ref[...]
        tr = tr_ref[...]
        cmp = (tr > tc) | ((tr == tc) & (cols < rows))
        acc_ref[...] += jnp.sum(cmp.astype(jnp.int32), axis=1, keepdims=True)

        @pl.when(j == nc - 1)
        def _():
            o_ref[...] = (acc_ref[...] < k).astype(jnp.int32)

    return pl.pallas_call(
        body,
        grid=(nb, nc),
        in_specs=[
            pl.BlockSpec((BN, 1), lambda i, j: (i, 0)),
            pl.BlockSpec((1, BC), lambda i, j: (0, j)),
        ],
        out_specs=pl.BlockSpec((BN, 1), lambda i, j: (i, 0)),
        out_shape=jax.ShapeDtypeStruct((Np, 1), jnp.int32),
        scratch_shapes=[pltpu.VMEM((BN, 1), jnp.int32)],
    )(t, tT)


def _cumsum_pos(kept):
    Np = kept.shape[0]
    nb = Np // BN

    def body(k_ref, o_ref, c_ref):
        i = pl.program_id(0)

        @pl.when(i == 0)
        def _():
            c_ref[0, 0] = 0

        kf = k_ref[...].astype(jnp.float32)
        r = lax.broadcasted_iota(jnp.int32, (BN, BN), 0)
        c = lax.broadcasted_iota(jnp.int32, (BN, BN), 1)
        tri = (r >= c).astype(jnp.float32)
        csum = jnp.dot(tri, kf, preferred_element_type=jnp.float32)
        o_ref[...] = csum.astype(jnp.int32) + c_ref[0, 0] - 1
        c_ref[0, 0] += jnp.sum(kf).astype(jnp.int32)

    return pl.pallas_call(
        body,
        grid=(nb,),
        in_specs=[pl.BlockSpec((BN, 1), lambda i: (i, 0))],
        out_specs=pl.BlockSpec((BN, 1), lambda i: (i, 0)),
        out_shape=jax.ShapeDtypeStruct((Np, 1), jnp.int32),
        scratch_shapes=[pltpu.SMEM((1, 1), jnp.int32)],
    )(kept)


def _deg_count(dst_t, Np, token=None):
    if token is not None:
        (dst_t,) = _sc_serial(token, dst_t)
    NW = dst_t.shape[1]
    rpt = Np // 16
    zrows = jnp.zeros((rpt, 128), jnp.float32)
    onesb = jnp.ones((WIN, 128), jnp.float32)
    mesh = plsc.VectorSubcoreMesh(core_axis_name="c", subcore_axis_name="s")

    @functools.partial(
        pl.kernel,
        mesh=mesh,
        out_type=jax.ShapeDtypeStruct((2 * Np, 128), jnp.float32),
        scratch_types=[
            pltpu.VMEM((1, NW, WIN), jnp.int32),
            pltpu.VMEM((WIN, 128), jnp.float32),
            pltpu.VMEM_SHARED((Np, 128), jnp.float32),
        ],
    )
    def k(dst_h, zr_h, ones_h, out_h, didx, onesv, acc):
        c = lax.axis_index("c")
        s = lax.axis_index("s")
        t = c * 16 + s
        lo = s * rpt
        pltpu.sync_copy(zr_h, acc.at[pl.ds(lo, rpt)])
        pltpu.sync_copy(dst_h.at[pl.ds(t, 1)], didx)
        pltpu.sync_copy(ones_h, onesv)
        plsc.subcore_barrier()
        for w in range(NW):
            pltpu.sync_copy(onesv, acc.at[didx.at[0, w]], add=True)
        plsc.subcore_barrier()
        pltpu.sync_copy(acc.at[pl.ds(lo, rpt)],
                        out_h.at[pl.ds(c * Np + lo, rpt)])

    return k(dst_t, zrows, onesb)


def _sc_serial(token, *arrs):
    out = lax.optimization_barrier((token, *arrs))
    return out[1:]


def _spmm(tab, src_t, dst_t, Np, D, Ds=None, token=None):
    if Ds is None:
        Ds = D
    if token is not None:
        tab, src_t, dst_t = _sc_serial(token, tab, src_t, dst_t)
    NW = src_t.shape[1]
    CH = 16
    NC = NW // CH
    rpt = Np // 16
    zrows = jnp.zeros((rpt, Ds), jnp.float32)
    mesh = plsc.VectorSubcoreMesh(core_axis_name="c", subcore_axis_name="s")

    @functools.partial(
        pl.kernel,
        mesh=mesh,
        out_type=jax.ShapeDtypeStruct((2 * Np, Ds), jnp.float32),
        scratch_types=[
            pltpu.VMEM((2, CH, WIN), jnp.int32),
            pltpu.VMEM((2, CH, WIN), jnp.int32),
            pltpu.VMEM((2, WIN, D), jnp.float32),
            pltpu.VMEM_SHARED((Np, Ds), jnp.float32),
            pltpu.SemaphoreType.DMA,
            pltpu.SemaphoreType.DMA,
        ],
    )
    def k(tab_h, src_h, dst_h, zr_h, out_h, sidx, didx, rowsb, acc, sem0, sem1):
        c = lax.axis_index("c")
        s = lax.axis_index("s")
        t = c * 16 + s
        lo = s * rpt
        pltpu.sync_copy(zr_h, acc.at[pl.ds(lo, rpt)])
        plsc.subcore_barrier()
        sems = (sem0, sem1)
        descs = [None, None]
        for ch in range(NC):
            p = ch % 2
            pltpu.sync_copy(src_h.at[t, pl.ds(ch * CH, CH)], sidx.at[p])
            pltpu.sync_copy(dst_h.at[t, pl.ds(ch * CH, CH)], didx.at[p])
            descs[0] = pltpu.async_copy(
                tab_h.at[sidx.at[p, 0]], rowsb.at[0], sems[0])
            for w in range(CH):
                cur = w % 2
                nxt = (w + 1) % 2
                descs[cur].wait()
                if w + 1 < CH:
                    descs[nxt] = pltpu.async_copy(
                        tab_h.at[sidx.at[p, w + 1]], rowsb.at[nxt], sems[nxt])
                if Ds == D:
                    srcslice = rowsb.at[cur]
                else:
                    srcslice = rowsb.at[cur, :, pl.ds(0, Ds)]
                pltpu.sync_copy(srcslice, acc.at[didx.at[p, w]], add=True)
        plsc.subcore_barrier()
        pltpu.sync_copy(acc.at[pl.ds(lo, rpt)],
                        out_h.at[pl.ds(c * Np + lo, rpt)])

    return k(tab, src_t, dst_t, zrows)


def _edges_col(src, dst, Np):
    sr = src.reshape(16, E_PAD // (16 * WIN), WIN)
    dr = dst.reshape(16, E_PAD // (16 * WIN), WIN)
    src_t = jnp.concatenate([sr, sr + Np], axis=0)
    dst_t = jnp.concatenate([dr, dr], axis=0)
    return src_t, dst_t


def _edges_split(src, dst):
    return (src.reshape(32, E_PAD // (32 * WIN), WIN),
            dst.reshape(32, E_PAD // (32 * WIN), WIN))


def kernel(x, edge_index, Wd0, bd0, Wd1, bd1, Wd2, bd2, Wd3, bd3,
           p1, p2, p3, Wu0, bu0, Wu1, bu1, Wu2, bu2, Wo, bo):
    f32 = jnp.float32
    N0, E = 10000, 160000
    Np0 = _pad_rows(N0)

    e = jnp.arange(E_PAD, dtype=jnp.int32)
    dead0 = (N0 + (e % SPREAD)).astype(jnp.int32)
    pad_e = E_PAD - E
    src0 = jnp.where(e < E, jnp.pad(edge_index[0], (0, pad_e)), dead0)
    dst0 = jnp.where(e < E, jnp.pad(edge_index[1], (0, pad_e)), dead0)

    xp = jnp.pad(x, ((0, Np0 - N0), (0, 0)))

    def graph_dinv(src, dst, Np, token):
        _, dt = _edges_split(src, dst)
        return _dinv_from_deg(_deg_count(dt, Np, token=token))

    def gcn_mid(zs2, src, dst, dinv, b, Np, n_valid, token):
        st, dt = _edges_col(src, dst, Np)
        y2 = _spmm(zs2, st, dt, Np, 128, token=token)
        return _post(y2, zs2, dinv, b, n_valid)

    dinv0 = graph_dinv(src0, dst0, Np0, None)
    inf0 = jnp.full((Np0, 1), jnp.inf, f32)
    zs2 = _mm_split(xp, Wd0, dinv0, inf0, N0)
    h = gcn_mid(zs2, src0, dst0, dinv0, bd0, Np0, N0, dinv0)

    Ns = [N0]
    Nps = [Np0]
    hs = [h]
    srcs = [src0]
    dsts = [dst0]
    dinvs = [dinv0]
    kepts, poss = [], []
    src, dst = src0, dst0
    n_val, Np = N0, Np0
    downW = [(Wd1, bd1), (Wd2, bd2), (Wd3, bd3)]
    ps = [p1, p2, p3]
    for i in range(3):
        kk = int(ceil(RATIOS[i] * n_val))
        t = _score(h, ps[i], n_val)
        kept = _rank_kept(t, kk)
        pos = _cumsum_pos(kept)
        N2 = kk
        Np2 = _pad_rows(N2)
        keptv = kept[:, 0].astype(bool)
        posv = pos[:, 0]
        ar = jnp.arange(Np, dtype=jnp.int32)
        sidx = jnp.where(keptv, posv, N2 + (ar % SPREAD))
        xn = jnp.zeros((Np2, 256), f32).at[sidx].set(h)
        tp = jnp.zeros((Np2, 1), f32).at[sidx].set(t)
        live = keptv[src] & keptv[dst]
        deade = (N2 + (e % SPREAD)).astype(jnp.int32)
        nsrc = jnp.where(live, posv[src], deade)
        ndst = jnp.where(live, posv[dst], deade)
        dinv = graph_dinv(nsrc, ndst, Np2, h)
        zs2 = _mm_split(xn, downW[i][0], dinv, tp, N2)
        h = gcn_mid(zs2, nsrc, ndst, dinv, downW[i][1], Np2, N2, dinv)
        kepts.append(kept)
        poss.append(pos)
        src, dst = nsrc, ndst
        n_val, Np = N2, Np2
        Ns.append(N2)
        Nps.append(Np2)
        if i < 2:
            hs.append(h)
            srcs.append(nsrc)
            dsts.append(ndst)
        dinvs.append(dinv)

    upW = [(Wu0, bu0), (Wu1, bu1), (Wu2, bu2)]
    for i in range(3):
        j = 2 - i
        res = hs[j]
        src, dst = srcs[j], dsts[j]
        keptv = kepts[j][:, 0].astype(bool)
        posv = poss[j][:, 0]
        n_val, Np = Ns[j], Nps[j]
        Npool = Ns[j + 1]
        dinv = dinvs[j]
        WA, WB = upW[i][0][:256], upW[i][0][256:]
        zh = _mm_plain(h, WB)
        ar = jnp.arange(Np, dtype=jnp.int32)
        uidx = jnp.where(keptv, posv, Npool + (ar % SPREAD))
        zup = zh[uidx] * keptv[:, None].astype(f32)
        zs2 = _mm_split_acc(res, WA, zup, dinv, n_val)
        h = gcn_mid(zs2, src, dst, dinv, upW[i][1], Np, n_val, h)

    Wo128 = jnp.pad(Wo, ((0, 0), (0, 88)))
    bo128 = jnp.pad(bo, (0, 88))
    zs128 = _mm_scaled(h, Wo128, dinv0, N0)
    st, dt = _edges_split(src0, dst0)
    y2 = _spmm(zs128, st, dt, Np0, 128, token=h)
    out = _post_final(y2, zs128, dinv0, bo128)
    return out[:N0, :40]
